# baseline (speedup 1.0000x reference)
import functools

import jax
import jax.numpy as jnp
from jax import lax
from jax.experimental import pallas as pl
from jax.experimental.pallas import tpu as pltpu
from jax.experimental.pallas import tpu_sc as plsc

NUM_SC = 2
NUM_SUBCORES = 16
LANES = 16
W = 16
NBUF = 4


def kernel(labels, train, force_drop_ids, embedding_table):
    del train
    B = labels.shape[0]
    V, D = embedding_table.shape
    NW = NUM_SC * NUM_SUBCORES
    b_per_w = B // NW
    n_blocks = b_per_w // W

    labels32 = labels.astype(jnp.int32)
    drops32 = force_drop_ids.astype(jnp.int32)

    mesh = plsc.VectorSubcoreMesh(core_axis_name="c", subcore_axis_name="s")

    scratch = (
        [pltpu.VMEM((b_per_w,), jnp.int32),
         pltpu.VMEM((b_per_w,), jnp.int32)]
        + [pltpu.VMEM((W, D), jnp.float32) for _ in range(NBUF)]
        + [pltpu.VMEM((W,), jnp.int32) for _ in range(NBUF)]
        + [pltpu.SemaphoreType.DMA for _ in range(2 * NBUF)]
    )

    @functools.partial(
        pl.kernel,
        mesh=mesh,
        out_type=jax.ShapeDtypeStruct((B, D), jnp.float32),
        scratch_types=scratch,
    )
    def emb(table_hbm, lab_hbm, fdi_hbm, out_hbm, lab_v, fdi_v, *bufs):
        rows = bufs[:NBUF]
        idxs = bufs[NBUF:2 * NBUF]
        gsems = bufs[2 * NBUF:3 * NBUF]
        osems = bufs[3 * NBUF:4 * NBUF]
        c = lax.axis_index("c")
        s = lax.axis_index("s")
        base = (c * NUM_SUBCORES + s) * b_per_w

        pltpu.sync_copy(lab_hbm.at[pl.ds(base, b_per_w)], lab_v)
        pltpu.sync_copy(fdi_hbm.at[pl.ds(base, b_per_w)], fdi_v)

        def prep_idx(r, idx_v):
            for h in range(W // LANES):
                hsl = pl.ds(r * W + h * LANES, LANES)
                idx_v[pl.ds(h * LANES, LANES)] = jnp.where(
                    fdi_v[hsl] == 1, V - 1, lab_v[hsl])

        def start_gather(b):
            pltpu.async_copy(table_hbm.at[idxs[b]], rows[b], gsems[b])

        def wait_gather(b):
            pltpu.make_async_copy(
                table_hbm.at[pl.ds(0, W)], rows[b], gsems[b]).wait()

        def start_out(r, b):
            pltpu.async_copy(
                rows[b], out_hbm.at[pl.ds(base + r * W, W)], osems[b])

        def wait_out(b):
            pltpu.make_async_copy(
                rows[b], out_hbm.at[pl.ds(0, W)], osems[b]).wait()

        for b in range(NBUF):
            prep_idx(b, idxs[b])
            start_gather(b)

        @pl.loop(0, n_blocks, step=NBUF)
        def _(rr):
            for b in range(NBUF):
                r = rr + b
                wait_gather(b)
                start_out(r, b)

                @pl.when(r + NBUF < n_blocks)
                def _():
                    wait_out(b)
                    prep_idx(r + NBUF, idxs[b])
                    start_gather(b)

        for b in range(NBUF):
            wait_out(b)

    return emb(embedding_table, labels32, drops32)
